# baseline (speedup 1.0000x reference)
import functools

import jax
import jax.numpy as jnp
import numpy as np
from jax import lax
from jax.experimental import pallas as pl
from jax.experimental.pallas import tpu as pltpu
from jax.experimental.pallas import tpu_sc as plsc

_N = 10000
_E = 320000
_H = 128
_NUM_TYPES = 100

_NC = 2
_NS = 16
_NW = _NC * _NS
_EPW = _E // _NW
_C = 80
_NCHUNK = _EPW // _C
_RPT = 624
_ZR = 48
_SEG = _H // 16

_EPERM = np.empty((_H,), dtype=np.int32)
for _b in range(_H // 32):
    for _k in range(16):
        _EPERM[32 * _b + 2 * _k] = 32 * _b + _k
        _EPERM[32 * _b + 2 * _k + 1] = 32 * _b + 16 + _k


def _sc_message(h, e, src, dst):
    mesh = plsc.VectorSubcoreMesh(core_axis_name="c", subcore_axis_name="s")

    @functools.partial(
        pl.kernel,
        out_type=jax.ShapeDtypeStruct((2 * _N, _H), jnp.float32),
        mesh=mesh,
        scratch_types=[
            pltpu.VMEM((_C,), jnp.int32),
            pltpu.VMEM((_C,), jnp.int32),
            pltpu.VMEM((_C,), jnp.int32),
            pltpu.VMEM((_C,), jnp.int32),
            pltpu.VMEM((_C, _H), jnp.float32),
            pltpu.VMEM((_C, _H), jnp.float32),
            pltpu.VMEM((_C, _H // 2), jnp.int32),
            pltpu.VMEM((_C, _H // 2), jnp.int32),
            pltpu.VMEM((_ZR, _H), jnp.float32),
            pltpu.VMEM_SHARED((_N, _H), jnp.float32),
            pltpu.SemaphoreType.DMA,
            pltpu.SemaphoreType.DMA,
            pltpu.SemaphoreType.DMA,
            pltpu.SemaphoreType.DMA,
            pltpu.SemaphoreType.DMA,
            pltpu.SemaphoreType.DMA,
        ],
    )
    def body(h_hbm, e_hbm, src_hbm, dst_hbm, out_hbm,
             srcA, srcB, dstA, dstB, hA, hB, eA, eB, zv, agg,
             semiA, semiB, semhA, semhB, semeA, semeB):
        cid = lax.axis_index("c")
        sid = lax.axis_index("s")
        wid = sid * _NC + cid
        base = wid * _EPW

        zero16 = jnp.zeros((16,), jnp.float32)

        def zero_row(r, carry):
            for j in range(_SEG):
                zv[r, pl.ds(j * 16, 16)] = zero16
            return carry

        lax.fori_loop(0, _ZR, zero_row, 0)
        for k in range(_RPT // _ZR):
            pltpu.sync_copy(zv, agg.at[pl.ds(sid * _RPT + k * _ZR, _ZR)])

        @pl.when(sid == _NS - 1)
        def _zero_tail():
            pltpu.sync_copy(zv.at[pl.ds(0, 16)], agg.at[pl.ds(_NS * _RPT, 16)])

        plsc.subcore_barrier()

        def issue_idx(c, srcv, dstv, sem_i):
            off = base + c * _C
            pltpu.async_copy(src_hbm.at[pl.ds(off, _C)], srcv, sem_i)
            pltpu.async_copy(dst_hbm.at[pl.ds(off, _C)], dstv, sem_i)

        def issue_gather(c, srcv, dstv, hv, ev, sem_i, sem_h, sem_e):
            pltpu.make_async_copy(src_hbm.at[pl.ds(0, _C)], srcv, sem_i).wait()
            pltpu.make_async_copy(dst_hbm.at[pl.ds(0, _C)], dstv, sem_i).wait()
            off = base + c * _C
            pltpu.async_copy(h_hbm.at[srcv], hv, sem_h)
            pltpu.async_copy(e_hbm.at[pl.ds(off, _C)], ev, sem_e)

        himask = jnp.full((16,), -65536, jnp.int32)
        shift16 = jnp.full((16,), 16, jnp.int32)

        def process(srcv, dstv, hv, ev, sem_h, sem_e):
            pltpu.make_async_copy(h_hbm.at[srcv], hv, sem_h).wait()
            pltpu.make_async_copy(e_hbm.at[pl.ds(0, _C)], ev, sem_e).wait()

            def one_row(r):
                for b in range(_H // 32):
                    ei = ev[r, pl.ds(16 * b, 16)]
                    elo = lax.bitcast_convert_type(
                        lax.shift_left(ei, shift16), jnp.float32)
                    ehi = lax.bitcast_convert_type(
                        lax.bitwise_and(ei, himask), jnp.float32)
                    s0 = pl.ds(32 * b, 16)
                    s1 = pl.ds(32 * b + 16, 16)
                    hv[r, s0] = jnp.maximum(hv[r, s0] + elo, 0.0)
                    hv[r, s1] = jnp.maximum(hv[r, s1] + ehi, 0.0)

            def row(r, c2):
                one_row(2 * r)
                one_row(2 * r + 1)
                return c2

            lax.fori_loop(0, _C // 2, row, 0)
            pltpu.sync_copy(hv, agg.at[dstv], add=True)

        issue_idx(0, srcA, dstA, semiA)
        issue_idx(1, srcB, dstB, semiB)
        issue_gather(0, srcA, dstA, hA, eA, semiA, semhA, semeA)

        def step(k, carry):
            c = 2 * k
            issue_gather(c + 1, srcB, dstB, hB, eB, semiB, semhB, semeB)
            process(srcA, dstA, hA, eA, semhA, semeA)
            issue_idx(c + 2, srcA, dstA, semiA)
            process(srcB, dstB, hB, eB, semhB, semeB)
            issue_gather(c + 2, srcA, dstA, hA, eA, semiA, semhA, semeA)

            @pl.when(c + 3 < _NCHUNK)
            def _prefetch_idx():
                issue_idx(c + 3, srcB, dstB, semiB)

            return carry

        lax.fori_loop(0, (_NCHUNK - 1) // 2, step, 0)
        process(srcA, dstA, hA, eA, semhA, semeA)
        plsc.subcore_barrier()

        for k in range(_RPT // _ZR):
            r0 = sid * _RPT + k * _ZR
            pltpu.sync_copy(agg.at[pl.ds(r0, _ZR)], zv)
            pltpu.sync_copy(zv, out_hbm.at[pl.ds(cid * _N + r0, _ZR)])

        @pl.when(sid == _NS - 1)
        def _read_tail():
            r0 = _NS * _RPT
            pltpu.sync_copy(agg.at[pl.ds(r0, 16)], zv.at[pl.ds(0, 16)])
            pltpu.sync_copy(zv.at[pl.ds(0, 16)], out_hbm.at[pl.ds(cid * _N + r0, 16)])

    return body(h, e, src, dst)


def _node_init_body(z_ref, x_ref, emb_ref, fw_ref, fb_ref, g_ref, b_ref, o_ref):
    z = z_ref[...]
    oh = (z == lax.broadcasted_iota(jnp.int32, (_N, _NUM_TYPES), 1))
    h = jnp.dot(oh.astype(jnp.float32), emb_ref[...],
                preferred_element_type=jnp.float32)
    h = h + jnp.dot(x_ref[...], fw_ref[...],
                    preferred_element_type=jnp.float32) + fb_ref[...]
    h = h * jax.nn.sigmoid(h)
    m = jnp.mean(h, axis=-1, keepdims=True)
    v = jnp.mean((h - m) * (h - m), axis=-1, keepdims=True)
    o_ref[...] = (h - m) * lax.rsqrt(v + 1e-5) * g_ref[...] + b_ref[...]


def _node_init(z, x, emb, feat_W, feat_b, ln0_g, ln0_b):
    return pl.pallas_call(
        _node_init_body,
        out_shape=jax.ShapeDtypeStruct((_N, _H), jnp.float32),
    )(z.reshape(_N, 1), x, emb, feat_W, feat_b.reshape(1, _H),
      ln0_g.reshape(1, _H), ln0_b.reshape(1, _H))


def _edge_init_body(ea_ref, w_ref, b_ref, o_ref):
    ea = ea_ref[...]
    w = w_ref[...]
    acc = b_ref[...]
    for k in range(4):
        acc = acc + ea[:, k:k + 1] * w[k:k + 1, :]
    o_ref[...] = (acc * jax.nn.sigmoid(acc)).astype(jnp.bfloat16)


def _edge_init(edge_attr, edge_W, edge_b):
    blk = 8000
    return pl.pallas_call(
        _edge_init_body,
        grid=(_E // blk,),
        in_specs=[
            pl.BlockSpec((blk, 4), lambda i: (i, 0)),
            pl.BlockSpec((4, _H), lambda i: (0, 0)),
            pl.BlockSpec((1, _H), lambda i: (0, 0)),
        ],
        out_specs=pl.BlockSpec((blk, _H), lambda i: (i, 0)),
        out_shape=jax.ShapeDtypeStruct((_E, _H), jnp.bfloat16),
    )(edge_attr, edge_W[:, _EPERM], edge_b[_EPERM].reshape(1, _H))


def _dense_body(h_ref, p0_ref, p1_ref, w1_ref, b1_ref, w2_ref, b2_ref,
                g_ref, b_ref, o_ref):
    h = h_ref[...]
    x0 = h + p0_ref[...] + p1_ref[...]
    t = jnp.dot(x0, w1_ref[...], preferred_element_type=jnp.float32) + b1_ref[...]
    t = t * jax.nn.sigmoid(t)
    t = jnp.dot(t, w2_ref[...], preferred_element_type=jnp.float32) + b2_ref[...]
    y = h + t
    m = jnp.mean(y, axis=-1, keepdims=True)
    v = jnp.mean((y - m) * (y - m), axis=-1, keepdims=True)
    o_ref[...] = (y - m) * lax.rsqrt(v + 1e-5) * g_ref[...] + b_ref[...]


def _dense_layer(h, p0, p1, W1, b1, W2, b2, g, b):
    blk = 2000
    return pl.pallas_call(
        _dense_body,
        grid=(_N // blk,),
        in_specs=[
            pl.BlockSpec((blk, _H), lambda i: (i, 0)),
            pl.BlockSpec((blk, _H), lambda i: (i, 0)),
            pl.BlockSpec((blk, _H), lambda i: (i, 0)),
            pl.BlockSpec((_H, _H), lambda i: (0, 0)),
            pl.BlockSpec((1, _H), lambda i: (0, 0)),
            pl.BlockSpec((_H, _H), lambda i: (0, 0)),
            pl.BlockSpec((1, _H), lambda i: (0, 0)),
            pl.BlockSpec((1, _H), lambda i: (0, 0)),
            pl.BlockSpec((1, _H), lambda i: (0, 0)),
        ],
        out_specs=pl.BlockSpec((blk, _H), lambda i: (i, 0)),
        out_shape=jax.ShapeDtypeStruct((_N, _H), jnp.float32),
    )(h, p0, p1, W1, b1.reshape(1, _H), W2, b2.reshape(1, _H),
      g.reshape(1, _H), b.reshape(1, _H))


def kernel(z, x, edge_index, edge_attr, batch_vec, emb, feat_W, feat_b,
           ln0_g, ln0_b, edge_W, edge_b, conv_W1, conv_b1, conv_W2, conv_b2,
           ln_g, ln_b):
    h = _node_init(z, x, emb, feat_W, feat_b, ln0_g, ln0_b)
    e = _edge_init(edge_attr, edge_W, edge_b)
    e = lax.bitcast_convert_type(e.reshape(_E, _H // 2, 2), jnp.int32)
    src = edge_index[0]
    dst = edge_index[1]
    for i in range(4):
        parts = _sc_message(h, e, src, dst)
        h = _dense_layer(h, parts[:_N], parts[_N:], conv_W1[i], conv_b1[i],
                         conv_W2[i], conv_b2[i], ln_g[i], ln_b[i])
    return (h, batch_vec)


# speedup vs baseline: 1.4572x; 1.4572x over previous
import functools

import jax
import jax.numpy as jnp
import numpy as np
from jax import lax
from jax.experimental import pallas as pl
from jax.experimental.pallas import tpu as pltpu
from jax.experimental.pallas import tpu_sc as plsc

_N = 10000
_E = 320000
_H = 128
_NUM_TYPES = 100

_NC = 2
_NS = 16
_NW = _NC * _NS
_EPW = _E // _NW
_C = 80
_NCHUNK = _EPW // _C
_RPT = 624
_ZR = 48
_SEG = _H // 16


def _sc_message(h, e, src, dst):
    mesh = plsc.VectorSubcoreMesh(core_axis_name="c", subcore_axis_name="s")

    @functools.partial(
        pl.kernel,
        out_type=jax.ShapeDtypeStruct((2 * _N, _H), jnp.float32),
        mesh=mesh,
        scratch_types=[
            pltpu.VMEM((_C,), jnp.int32),
            pltpu.VMEM((_C,), jnp.int32),
            pltpu.VMEM((_C,), jnp.int32),
            pltpu.VMEM((_C,), jnp.int32),
            pltpu.VMEM((_C, _H), jnp.float32),
            pltpu.VMEM((_C, _H), jnp.float32),
            pltpu.VMEM((_C, _H // 2), jnp.int32),
            pltpu.VMEM((_C, _H // 2), jnp.int32),
            pltpu.VMEM((_ZR, _H), jnp.float32),
            pltpu.VMEM_SHARED((_N, _H), jnp.float32),
            pltpu.SemaphoreType.DMA,
            pltpu.SemaphoreType.DMA,
            pltpu.SemaphoreType.DMA,
            pltpu.SemaphoreType.DMA,
            pltpu.SemaphoreType.DMA,
            pltpu.SemaphoreType.DMA,
        ],
    )
    def body(h_hbm, e_hbm, src_hbm, dst_hbm, out_hbm,
             srcA, srcB, dstA, dstB, hA, hB, eA, eB, zv, agg,
             semiA, semiB, semhA, semhB, semeA, semeB):
        cid = lax.axis_index("c")
        sid = lax.axis_index("s")
        wid = sid * _NC + cid
        base = wid * _EPW

        zero16 = jnp.zeros((16,), jnp.float32)

        def zero_row(r, carry):
            for j in range(_SEG):
                zv[r, pl.ds(j * 16, 16)] = zero16
            return carry

        lax.fori_loop(0, _ZR, zero_row, 0)
        for k in range(_RPT // _ZR):
            pltpu.sync_copy(zv, agg.at[pl.ds(sid * _RPT + k * _ZR, _ZR)])

        @pl.when(sid == _NS - 1)
        def _zero_tail():
            pltpu.sync_copy(zv.at[pl.ds(0, 16)], agg.at[pl.ds(_NS * _RPT, 16)])

        plsc.subcore_barrier()

        def issue_idx(c, srcv, dstv, sem_i):
            off = base + c * _C
            pltpu.async_copy(src_hbm.at[pl.ds(off, _C)], srcv, sem_i)
            pltpu.async_copy(dst_hbm.at[pl.ds(off, _C)], dstv, sem_i)

        def issue_gather(c, srcv, dstv, hv, ev, sem_i, sem_h, sem_e):
            pltpu.make_async_copy(src_hbm.at[pl.ds(0, _C)], srcv, sem_i).wait()
            pltpu.make_async_copy(dst_hbm.at[pl.ds(0, _C)], dstv, sem_i).wait()
            off = base + c * _C
            pltpu.async_copy(h_hbm.at[srcv], hv, sem_h)
            pltpu.async_copy(e_hbm.at[pl.ds(off, _C)], ev, sem_e)

        himask = jnp.full((16,), -65536, jnp.int32)
        shift16 = jnp.full((16,), 16, jnp.int32)

        def process(srcv, dstv, hv, ev, sem_h, sem_e):
            pltpu.make_async_copy(h_hbm.at[srcv], hv, sem_h).wait()
            pltpu.make_async_copy(e_hbm.at[pl.ds(0, _C)], ev, sem_e).wait()

            def one_row(r):
                for b in range(_H // 32):
                    ei = ev[r, pl.ds(16 * b, 16)]
                    elo = lax.bitcast_convert_type(
                        lax.shift_left(ei, shift16), jnp.float32)
                    ehi = lax.bitcast_convert_type(
                        lax.bitwise_and(ei, himask), jnp.float32)
                    s0 = pl.ds(32 * b, 16)
                    s1 = pl.ds(32 * b + 16, 16)
                    hv[r, s0] = jnp.maximum(hv[r, s0] + elo, 0.0)
                    hv[r, s1] = jnp.maximum(hv[r, s1] + ehi, 0.0)

            def row(r, c2):
                one_row(2 * r)
                one_row(2 * r + 1)
                return c2

            lax.fori_loop(0, _C // 2, row, 0)
            pltpu.sync_copy(hv, agg.at[dstv], add=True)

        issue_idx(0, srcA, dstA, semiA)
        issue_idx(1, srcB, dstB, semiB)
        issue_gather(0, srcA, dstA, hA, eA, semiA, semhA, semeA)

        def step(k, carry):
            c = 2 * k
            issue_gather(c + 1, srcB, dstB, hB, eB, semiB, semhB, semeB)
            process(srcA, dstA, hA, eA, semhA, semeA)
            issue_idx(c + 2, srcA, dstA, semiA)
            process(srcB, dstB, hB, eB, semhB, semeB)
            issue_gather(c + 2, srcA, dstA, hA, eA, semiA, semhA, semeA)

            @pl.when(c + 3 < _NCHUNK)
            def _prefetch_idx():
                issue_idx(c + 3, srcB, dstB, semiB)

            return carry

        lax.fori_loop(0, (_NCHUNK - 1) // 2, step, 0)
        process(srcA, dstA, hA, eA, semhA, semeA)
        plsc.subcore_barrier()

        for k in range(_RPT // _ZR):
            r0 = sid * _RPT + k * _ZR
            pltpu.sync_copy(agg.at[pl.ds(r0, _ZR)], zv)
            pltpu.sync_copy(zv, out_hbm.at[pl.ds(cid * _N + r0, _ZR)])

        @pl.when(sid == _NS - 1)
        def _read_tail():
            r0 = _NS * _RPT
            pltpu.sync_copy(agg.at[pl.ds(r0, 16)], zv.at[pl.ds(0, 16)])
            pltpu.sync_copy(zv.at[pl.ds(0, 16)], out_hbm.at[pl.ds(cid * _N + r0, 16)])

    return body(h, e, src, dst)


def _node_init_body(z_ref, x_ref, emb_ref, fw_ref, fb_ref, g_ref, b_ref, o_ref):
    z = z_ref[...]
    oh = (z == lax.broadcasted_iota(jnp.int32, (_N, _NUM_TYPES), 1))
    h = jnp.dot(oh.astype(jnp.float32), emb_ref[...],
                preferred_element_type=jnp.float32)
    h = h + jnp.dot(x_ref[...], fw_ref[...],
                    preferred_element_type=jnp.float32) + fb_ref[...]
    h = h * jax.nn.sigmoid(h)
    m = jnp.mean(h, axis=-1, keepdims=True)
    v = jnp.mean((h - m) * (h - m), axis=-1, keepdims=True)
    o_ref[...] = (h - m) * lax.rsqrt(v + 1e-5) * g_ref[...] + b_ref[...]


def _node_init(z, x, emb, feat_W, feat_b, ln0_g, ln0_b):
    return pl.pallas_call(
        _node_init_body,
        out_shape=jax.ShapeDtypeStruct((_N, _H), jnp.float32),
    )(z.reshape(_N, 1), x, emb, feat_W, feat_b.reshape(1, _H),
      ln0_g.reshape(1, _H), ln0_b.reshape(1, _H))


def _edge_init_body(ea_ref, w_ref, b_ref, o_ref):
    ea = ea_ref[...]
    w = w_ref[...]
    acc = b_ref[...]
    for k in range(4):
        acc = acc + ea[:, k:k + 1] * w[k:k + 1, :]
    acc = acc * jax.nn.sigmoid(acc)
    rb = lax.bitcast_convert_type(acc, jnp.int32) + 0x8000
    cols = []
    for b in range(_H // 32):
        lo = lax.shift_right_logical(rb[:, 32 * b:32 * b + 16], 16)
        hi = lax.bitwise_and(rb[:, 32 * b + 16:32 * b + 32], -65536)
        cols.append(lax.bitwise_or(lo, hi))
    o_ref[...] = jnp.concatenate(cols, axis=1)


def _edge_init(edge_attr, edge_W, edge_b):
    blk = 8000
    return pl.pallas_call(
        _edge_init_body,
        grid=(_E // blk,),
        in_specs=[
            pl.BlockSpec((blk, 4), lambda i: (i, 0)),
            pl.BlockSpec((4, _H), lambda i: (0, 0)),
            pl.BlockSpec((1, _H), lambda i: (0, 0)),
        ],
        out_specs=pl.BlockSpec((blk, _H // 2), lambda i: (i, 0)),
        out_shape=jax.ShapeDtypeStruct((_E, _H // 2), jnp.int32),
    )(edge_attr, edge_W, edge_b.reshape(1, _H))


def _dense_body(h_ref, p0_ref, p1_ref, w1_ref, b1_ref, w2_ref, b2_ref,
                g_ref, b_ref, o_ref):
    h = h_ref[...]
    x0 = h + p0_ref[...] + p1_ref[...]
    t = jnp.dot(x0, w1_ref[...], preferred_element_type=jnp.float32) + b1_ref[...]
    t = t * jax.nn.sigmoid(t)
    t = jnp.dot(t, w2_ref[...], preferred_element_type=jnp.float32) + b2_ref[...]
    y = h + t
    m = jnp.mean(y, axis=-1, keepdims=True)
    v = jnp.mean((y - m) * (y - m), axis=-1, keepdims=True)
    o_ref[...] = (y - m) * lax.rsqrt(v + 1e-5) * g_ref[...] + b_ref[...]


def _dense_layer(h, p0, p1, W1, b1, W2, b2, g, b):
    blk = 2000
    return pl.pallas_call(
        _dense_body,
        grid=(_N // blk,),
        in_specs=[
            pl.BlockSpec((blk, _H), lambda i: (i, 0)),
            pl.BlockSpec((blk, _H), lambda i: (i, 0)),
            pl.BlockSpec((blk, _H), lambda i: (i, 0)),
            pl.BlockSpec((_H, _H), lambda i: (0, 0)),
            pl.BlockSpec((1, _H), lambda i: (0, 0)),
            pl.BlockSpec((_H, _H), lambda i: (0, 0)),
            pl.BlockSpec((1, _H), lambda i: (0, 0)),
            pl.BlockSpec((1, _H), lambda i: (0, 0)),
            pl.BlockSpec((1, _H), lambda i: (0, 0)),
        ],
        out_specs=pl.BlockSpec((blk, _H), lambda i: (i, 0)),
        out_shape=jax.ShapeDtypeStruct((_N, _H), jnp.float32),
    )(h, p0, p1, W1, b1.reshape(1, _H), W2, b2.reshape(1, _H),
      g.reshape(1, _H), b.reshape(1, _H))


def kernel(z, x, edge_index, edge_attr, batch_vec, emb, feat_W, feat_b,
           ln0_g, ln0_b, edge_W, edge_b, conv_W1, conv_b1, conv_W2, conv_b2,
           ln_g, ln_b):
    h = _node_init(z, x, emb, feat_W, feat_b, ln0_g, ln0_b)
    e = _edge_init(edge_attr, edge_W, edge_b)
    src = edge_index[0]
    dst = edge_index[1]
    for i in range(4):
        parts = _sc_message(h, e, src, dst)
        h = _dense_layer(h, parts[:_N], parts[_N:], conv_W1[i], conv_b1[i],
                         conv_W2[i], conv_b2[i], ln_g[i], ln_b[i])
    return (h, batch_vec)


# speedup vs baseline: 1.7130x; 1.1756x over previous
import functools

import jax
import jax.numpy as jnp
import numpy as np
from jax import lax
from jax.experimental import pallas as pl
from jax.experimental.pallas import tpu as pltpu
from jax.experimental.pallas import tpu_sc as plsc

_N = 10000
_E = 320000
_H = 128
_NUM_TYPES = 100

_NC = 2
_NS = 16
_NW = _NC * _NS
_EPW = _E // _NW
_C = 80
_NCHUNK = _EPW // _C
_RPT = 624
_ZR = 48
_SEG = _H // 16


def _sc_message(h, e, src, dst):
    mesh = plsc.VectorSubcoreMesh(core_axis_name="c", subcore_axis_name="s")

    @functools.partial(
        pl.kernel,
        out_type=jax.ShapeDtypeStruct((2 * _N, _H), jnp.float32),
        mesh=mesh,
        scratch_types=[
            pltpu.VMEM((_C,), jnp.int32),
            pltpu.VMEM((_C,), jnp.int32),
            pltpu.VMEM((_C,), jnp.int32),
            pltpu.VMEM((_C,), jnp.int32),
            pltpu.VMEM((_C, _H), jnp.float32),
            pltpu.VMEM((_C, _H), jnp.float32),
            pltpu.VMEM((_C, _H // 2), jnp.int32),
            pltpu.VMEM((_C, _H // 2), jnp.int32),
            pltpu.VMEM((_ZR, _H), jnp.float32),
            pltpu.VMEM_SHARED((_N, _H), jnp.float32),
            pltpu.SemaphoreType.DMA,
            pltpu.SemaphoreType.DMA,
            pltpu.SemaphoreType.DMA,
            pltpu.SemaphoreType.DMA,
            pltpu.SemaphoreType.DMA,
            pltpu.SemaphoreType.DMA,
        ],
    )
    def body(h_hbm, e_hbm, src_hbm, dst_hbm, out_hbm,
             srcA, srcB, dstA, dstB, hA, hB, eA, eB, zv, agg,
             semiA, semiB, semhA, semhB, semeA, semeB):
        cid = lax.axis_index("c")
        sid = lax.axis_index("s")
        wid = sid * _NC + cid
        base = wid * _EPW

        zero16 = jnp.zeros((16,), jnp.float32)

        def zero_row(r, carry):
            for j in range(_SEG):
                zv[r, pl.ds(j * 16, 16)] = zero16
            return carry

        lax.fori_loop(0, _ZR, zero_row, 0)
        for k in range(_RPT // _ZR):
            pltpu.sync_copy(zv, agg.at[pl.ds(sid * _RPT + k * _ZR, _ZR)])

        @pl.when(sid == _NS - 1)
        def _zero_tail():
            pltpu.sync_copy(zv.at[pl.ds(0, 16)], agg.at[pl.ds(_NS * _RPT, 16)])

        plsc.subcore_barrier()

        def issue_idx(c, srcv, dstv, sem_i):
            off = base + c * _C
            pltpu.async_copy(src_hbm.at[pl.ds(off, _C)], srcv, sem_i)
            pltpu.async_copy(dst_hbm.at[pl.ds(off, _C)], dstv, sem_i)

        def issue_gather(c, srcv, dstv, hv, ev, sem_i, sem_h, sem_e):
            pltpu.make_async_copy(src_hbm.at[pl.ds(0, _C)], srcv, sem_i).wait()
            pltpu.make_async_copy(dst_hbm.at[pl.ds(0, _C)], dstv, sem_i).wait()
            off = base + c * _C
            pltpu.async_copy(h_hbm.at[srcv], hv, sem_h)
            pltpu.async_copy(e_hbm.at[pl.ds(off, _C)], ev, sem_e)

        himask = jnp.full((16,), -65536, jnp.int32)
        shift16 = jnp.full((16,), 16, jnp.int32)

        def process(srcv, dstv, hv, ev, sem_h, sem_e):
            pltpu.make_async_copy(h_hbm.at[srcv], hv, sem_h).wait()
            pltpu.make_async_copy(e_hbm.at[pl.ds(0, _C)], ev, sem_e).wait()

            def one_row(r):
                for b in range(_H // 32):
                    ei = ev[r, pl.ds(16 * b, 16)]
                    elo = lax.bitcast_convert_type(
                        lax.shift_left(ei, shift16), jnp.float32)
                    ehi = lax.bitcast_convert_type(
                        lax.bitwise_and(ei, himask), jnp.float32)
                    s0 = pl.ds(16 * b, 16)
                    s1 = pl.ds(64 + 16 * b, 16)
                    hv[r, s0] = jnp.maximum(hv[r, s0] + elo, 0.0)
                    hv[r, s1] = jnp.maximum(hv[r, s1] + ehi, 0.0)

            def row(r, c2):
                one_row(2 * r)
                one_row(2 * r + 1)
                return c2

            lax.fori_loop(0, _C // 2, row, 0)
            pltpu.sync_copy(hv, agg.at[dstv], add=True)

        issue_idx(0, srcA, dstA, semiA)
        issue_idx(1, srcB, dstB, semiB)
        issue_gather(0, srcA, dstA, hA, eA, semiA, semhA, semeA)

        def step(k, carry):
            c = 2 * k
            issue_gather(c + 1, srcB, dstB, hB, eB, semiB, semhB, semeB)
            process(srcA, dstA, hA, eA, semhA, semeA)
            issue_idx(c + 2, srcA, dstA, semiA)
            process(srcB, dstB, hB, eB, semhB, semeB)
            issue_gather(c + 2, srcA, dstA, hA, eA, semiA, semhA, semeA)

            @pl.when(c + 3 < _NCHUNK)
            def _prefetch_idx():
                issue_idx(c + 3, srcB, dstB, semiB)

            return carry

        lax.fori_loop(0, (_NCHUNK - 1) // 2, step, 0)
        process(srcA, dstA, hA, eA, semhA, semeA)
        plsc.subcore_barrier()

        for k in range(_RPT // _ZR):
            r0 = sid * _RPT + k * _ZR
            pltpu.sync_copy(agg.at[pl.ds(r0, _ZR)], zv)
            pltpu.sync_copy(zv, out_hbm.at[pl.ds(cid * _N + r0, _ZR)])

        @pl.when(sid == _NS - 1)
        def _read_tail():
            r0 = _NS * _RPT
            pltpu.sync_copy(agg.at[pl.ds(r0, 16)], zv.at[pl.ds(0, 16)])
            pltpu.sync_copy(zv.at[pl.ds(0, 16)], out_hbm.at[pl.ds(cid * _N + r0, 16)])

    return body(h, e, src, dst)


def _node_init_body(z_ref, x_ref, emb_ref, fw_ref, fb_ref, g_ref, b_ref, o_ref):
    z = z_ref[...]
    oh = (z == lax.broadcasted_iota(jnp.int32, (_N, _NUM_TYPES), 1))
    h = jnp.dot(oh.astype(jnp.float32), emb_ref[...],
                preferred_element_type=jnp.float32)
    h = h + jnp.dot(x_ref[...], fw_ref[...],
                    preferred_element_type=jnp.float32) + fb_ref[...]
    h = h * jax.nn.sigmoid(h)
    m = jnp.mean(h, axis=-1, keepdims=True)
    v = jnp.mean((h - m) * (h - m), axis=-1, keepdims=True)
    o_ref[...] = (h - m) * lax.rsqrt(v + 1e-5) * g_ref[...] + b_ref[...]


def _node_init(z, x, emb, feat_W, feat_b, ln0_g, ln0_b):
    return pl.pallas_call(
        _node_init_body,
        out_shape=jax.ShapeDtypeStruct((_N, _H), jnp.float32),
    )(z.reshape(_N, 1), x, emb, feat_W, feat_b.reshape(1, _H),
      ln0_g.reshape(1, _H), ln0_b.reshape(1, _H))


def _edge_init_body(ea_ref, w_ref, b_ref, o_ref):
    acc = jnp.dot(ea_ref[...], w_ref[...],
                  preferred_element_type=jnp.float32) + b_ref[...]
    acc = acc * jax.nn.sigmoid(acc)
    rb = lax.bitcast_convert_type(acc, jnp.int32) + 0x8000
    o_ref[...] = lax.bitwise_or(
        lax.shift_right_logical(rb[:, :_H // 2], 16),
        lax.bitwise_and(rb[:, _H // 2:], -65536))


def _edge_init(edge_attr, edge_W, edge_b):
    blk = 8000
    return pl.pallas_call(
        _edge_init_body,
        grid=(_E // blk,),
        in_specs=[
            pl.BlockSpec((blk, 4), lambda i: (i, 0)),
            pl.BlockSpec((4, _H), lambda i: (0, 0)),
            pl.BlockSpec((1, _H), lambda i: (0, 0)),
        ],
        out_specs=pl.BlockSpec((blk, _H // 2), lambda i: (i, 0)),
        out_shape=jax.ShapeDtypeStruct((_E, _H // 2), jnp.int32),
    )(edge_attr, edge_W, edge_b.reshape(1, _H))


def _dense_body(h_ref, p0_ref, p1_ref, w1_ref, b1_ref, w2_ref, b2_ref,
                g_ref, b_ref, o_ref):
    h = h_ref[...]
    x0 = h + p0_ref[...] + p1_ref[...]
    t = jnp.dot(x0, w1_ref[...], preferred_element_type=jnp.float32) + b1_ref[...]
    t = t * jax.nn.sigmoid(t)
    t = jnp.dot(t, w2_ref[...], preferred_element_type=jnp.float32) + b2_ref[...]
    y = h + t
    m = jnp.mean(y, axis=-1, keepdims=True)
    v = jnp.mean((y - m) * (y - m), axis=-1, keepdims=True)
    o_ref[...] = (y - m) * lax.rsqrt(v + 1e-5) * g_ref[...] + b_ref[...]


def _dense_layer(h, p0, p1, W1, b1, W2, b2, g, b):
    blk = 2000
    return pl.pallas_call(
        _dense_body,
        grid=(_N // blk,),
        in_specs=[
            pl.BlockSpec((blk, _H), lambda i: (i, 0)),
            pl.BlockSpec((blk, _H), lambda i: (i, 0)),
            pl.BlockSpec((blk, _H), lambda i: (i, 0)),
            pl.BlockSpec((_H, _H), lambda i: (0, 0)),
            pl.BlockSpec((1, _H), lambda i: (0, 0)),
            pl.BlockSpec((_H, _H), lambda i: (0, 0)),
            pl.BlockSpec((1, _H), lambda i: (0, 0)),
            pl.BlockSpec((1, _H), lambda i: (0, 0)),
            pl.BlockSpec((1, _H), lambda i: (0, 0)),
        ],
        out_specs=pl.BlockSpec((blk, _H), lambda i: (i, 0)),
        out_shape=jax.ShapeDtypeStruct((_N, _H), jnp.float32),
    )(h, p0, p1, W1, b1.reshape(1, _H), W2, b2.reshape(1, _H),
      g.reshape(1, _H), b.reshape(1, _H))


def kernel(z, x, edge_index, edge_attr, batch_vec, emb, feat_W, feat_b,
           ln0_g, ln0_b, edge_W, edge_b, conv_W1, conv_b1, conv_W2, conv_b2,
           ln_g, ln_b):
    h = _node_init(z, x, emb, feat_W, feat_b, ln0_g, ln0_b)
    e = _edge_init(edge_attr, edge_W, edge_b)
    src = edge_index[0]
    dst = edge_index[1]
    for i in range(4):
        parts = _sc_message(h, e, src, dst)
        h = _dense_layer(h, parts[:_N], parts[_N:], conv_W1[i], conv_b1[i],
                         conv_W2[i], conv_b2[i], ln_g[i], ln_b[i])
    return (h, batch_vec)


# speedup vs baseline: 1.7182x; 1.0031x over previous
import functools

import jax
import jax.numpy as jnp
import numpy as np
from jax import lax
from jax.experimental import pallas as pl
from jax.experimental.pallas import tpu as pltpu
from jax.experimental.pallas import tpu_sc as plsc

_N = 10000
_E = 320000
_H = 128
_NUM_TYPES = 100

_NC = 2
_NS = 16
_NW = _NC * _NS
_EPW = _E // _NW
_C = 80
_NCHUNK = _EPW // _C
_RPT = 624
_ZR = 48
_SEG = _H // 16


def _sc_message(h, e, src, dst):
    mesh = plsc.VectorSubcoreMesh(core_axis_name="c", subcore_axis_name="s")

    @functools.partial(
        pl.kernel,
        out_type=jax.ShapeDtypeStruct((2 * _N, _H), jnp.float32),
        mesh=mesh,
        scratch_types=[
            pltpu.VMEM((_C,), jnp.int32),
            pltpu.VMEM((_C,), jnp.int32),
            pltpu.VMEM((_C,), jnp.int32),
            pltpu.VMEM((_C,), jnp.int32),
            pltpu.VMEM((_C, _H), jnp.float32),
            pltpu.VMEM((_C, _H), jnp.float32),
            pltpu.VMEM((_C, _H // 2), jnp.int32),
            pltpu.VMEM((_C, _H), jnp.float32),
            pltpu.VMEM((_C,), jnp.int32),
            pltpu.VMEM_SHARED((_N, _H), jnp.float32),
            pltpu.SemaphoreType.DMA,
            pltpu.SemaphoreType.DMA,
            pltpu.SemaphoreType.DMA,
            pltpu.SemaphoreType.DMA,
            pltpu.SemaphoreType.DMA,
            pltpu.SemaphoreType.DMA,
        ],
    )
    def body(h_hbm, e_hbm, src_hbm, dst_hbm, out_hbm,
             srcA, srcB, dstA, dstB, hA, hB, ev, mv, dsts, agg,
             semiA, semiB, semhA, semhB, seme, sems):
        cid = lax.axis_index("c")
        sid = lax.axis_index("s")
        wid = sid * _NC + cid
        base = wid * _EPW

        zero16 = jnp.zeros((16,), jnp.float32)

        def zero_row(r, carry):
            for j in range(_SEG):
                mv[r, pl.ds(j * 16, 16)] = zero16
            return carry

        lax.fori_loop(0, _ZR, zero_row, 0)
        for k in range(_RPT // _ZR):
            pltpu.sync_copy(mv.at[pl.ds(0, _ZR)],
                            agg.at[pl.ds(sid * _RPT + k * _ZR, _ZR)])

        @pl.when(sid == _NS - 1)
        def _zero_tail():
            pltpu.sync_copy(mv.at[pl.ds(0, 16)], agg.at[pl.ds(_NS * _RPT, 16)])

        plsc.subcore_barrier()

        def issue_idx(c, srcv, dstv, sem_i):
            off = base + c * _C
            pltpu.async_copy(src_hbm.at[pl.ds(off, _C)], srcv, sem_i)
            pltpu.async_copy(dst_hbm.at[pl.ds(off, _C)], dstv, sem_i)

        def issue_e(c):
            pltpu.async_copy(e_hbm.at[pl.ds(base + c * _C, _C)], ev, seme)

        def issue_gather(c, srcv, dstv, hv, sem_i, sem_h):
            pltpu.make_async_copy(src_hbm.at[pl.ds(0, _C)], srcv, sem_i).wait()
            pltpu.make_async_copy(dst_hbm.at[pl.ds(0, _C)], dstv, sem_i).wait()
            pltpu.async_copy(h_hbm.at[srcv], hv, sem_h)

        himask = jnp.full((16,), -65536, jnp.int32)
        shift16 = jnp.full((16,), 16, jnp.int32)

        def process(c, srcv, dstv, hv, sem_h, wait_prev):
            pltpu.make_async_copy(h_hbm.at[srcv], hv, sem_h).wait()
            pltpu.make_async_copy(e_hbm.at[pl.ds(0, _C)], ev, seme).wait()
            if wait_prev:
                pltpu.make_async_copy(mv, agg.at[dsts], sems).wait()
            for i in range(_C // 16):
                dsts[pl.ds(16 * i, 16)] = dstv[pl.ds(16 * i, 16)]

            def one_row(r):
                for b in range(_H // 32):
                    s = pl.ds(16 * b, 16)
                    ei = ev[r, s]
                    elo = lax.bitcast_convert_type(
                        lax.shift_left(ei, shift16), jnp.float32)
                    ehi = lax.bitcast_convert_type(
                        lax.bitwise_and(ei, himask), jnp.float32)
                    s1 = pl.ds(64 + 16 * b, 16)
                    mv[r, s] = jnp.maximum(hv[r, s] + elo, 0.0)
                    mv[r, s1] = jnp.maximum(hv[r, s1] + ehi, 0.0)

            def row(r, c2):
                one_row(2 * r)
                one_row(2 * r + 1)
                return c2

            lax.fori_loop(0, _C // 2, row, 0)

            @pl.when(c + 1 < _NCHUNK)
            def _next_e():
                issue_e(c + 1)

            pltpu.async_copy(mv, agg.at[dsts], sems, add=True)

        issue_idx(0, srcA, dstA, semiA)
        issue_idx(1, srcB, dstB, semiB)
        issue_e(0)
        issue_gather(0, srcA, dstA, hA, semiA, semhA)
        issue_gather(1, srcB, dstB, hB, semiB, semhB)
        process(0, srcA, dstA, hA, semhA, False)
        issue_idx(2, srcA, dstA, semiA)

        def step(k, carry):
            c = 2 * k
            issue_gather(c + 2, srcA, dstA, hA, semiA, semhA)
            process(c + 1, srcB, dstB, hB, semhB, True)

            @pl.when(c + 3 < _NCHUNK)
            def _idx_b():
                issue_idx(c + 3, srcB, dstB, semiB)

            process(c + 2, srcA, dstA, hA, semhA, True)

            @pl.when(c + 3 < _NCHUNK)
            def _gather_b():
                issue_gather(c + 3, srcB, dstB, hB, semiB, semhB)

            @pl.when(c + 4 < _NCHUNK)
            def _idx_a():
                issue_idx(c + 4, srcA, dstA, semiA)

            return carry

        lax.fori_loop(0, (_NCHUNK - 1) // 2, step, 0)
        pltpu.make_async_copy(mv, agg.at[dsts], sems).wait()
        plsc.subcore_barrier()

        for k in range(_RPT // _ZR):
            r0 = sid * _RPT + k * _ZR
            pltpu.sync_copy(agg.at[pl.ds(r0, _ZR)], mv.at[pl.ds(0, _ZR)])
            pltpu.sync_copy(mv.at[pl.ds(0, _ZR)],
                            out_hbm.at[pl.ds(cid * _N + r0, _ZR)])

        @pl.when(sid == _NS - 1)
        def _read_tail():
            r0 = _NS * _RPT
            pltpu.sync_copy(agg.at[pl.ds(r0, 16)], mv.at[pl.ds(0, 16)])
            pltpu.sync_copy(mv.at[pl.ds(0, 16)],
                            out_hbm.at[pl.ds(cid * _N + r0, 16)])

    return body(h, e, src, dst)


def _pack_bf16_pairs(h):
    rb = lax.bitcast_convert_type(h, jnp.int32) + 0x8000
    return lax.bitwise_or(
        lax.shift_right_logical(rb[:, :_H // 2], 16),
        lax.bitwise_and(rb[:, _H // 2:], -65536))


def _node_init_body(z_ref, x_ref, emb_ref, fw_ref, fb_ref, g_ref, b_ref,
                    o_ref):
    z = z_ref[...]
    oh = (z == lax.broadcasted_iota(jnp.int32, (_N, _NUM_TYPES), 1))
    h = jnp.dot(oh.astype(jnp.float32), emb_ref[...],
                preferred_element_type=jnp.float32)
    h = h + jnp.dot(x_ref[...], fw_ref[...],
                    preferred_element_type=jnp.float32) + fb_ref[...]
    h = h * jax.nn.sigmoid(h)
    m = jnp.mean(h, axis=-1, keepdims=True)
    v = jnp.mean((h - m) * (h - m), axis=-1, keepdims=True)
    o_ref[...] = (h - m) * lax.rsqrt(v + 1e-5) * g_ref[...] + b_ref[...]


def _node_init(z, x, emb, feat_W, feat_b, ln0_g, ln0_b):
    return pl.pallas_call(
        _node_init_body,
        out_shape=jax.ShapeDtypeStruct((_N, _H), jnp.float32),
    )(z.reshape(_N, 1), x, emb, feat_W, feat_b.reshape(1, _H),
      ln0_g.reshape(1, _H), ln0_b.reshape(1, _H))


def _edge_init_body(ea_ref, w_ref, b_ref, o_ref):
    acc = jnp.dot(ea_ref[...], w_ref[...],
                  preferred_element_type=jnp.float32) + b_ref[...]
    acc = acc * jax.nn.sigmoid(acc)
    o_ref[...] = _pack_bf16_pairs(acc)


def _edge_init(edge_attr, edge_W, edge_b):
    blk = 8000
    return pl.pallas_call(
        _edge_init_body,
        grid=(_E // blk,),
        in_specs=[
            pl.BlockSpec((blk, 4), lambda i: (i, 0)),
            pl.BlockSpec((4, _H), lambda i: (0, 0)),
            pl.BlockSpec((1, _H), lambda i: (0, 0)),
        ],
        out_specs=pl.BlockSpec((blk, _H // 2), lambda i: (i, 0)),
        out_shape=jax.ShapeDtypeStruct((_E, _H // 2), jnp.int32),
    )(edge_attr, edge_W, edge_b.reshape(1, _H))


def _dense_body(h_ref, p0_ref, p1_ref, w1_ref, b1_ref, w2_ref, b2_ref,
                g_ref, b_ref, o_ref):
    h = h_ref[...]
    x0 = h + p0_ref[...] + p1_ref[...]
    t = jnp.dot(x0, w1_ref[...], preferred_element_type=jnp.float32) + b1_ref[...]
    t = t * jax.nn.sigmoid(t)
    t = jnp.dot(t, w2_ref[...], preferred_element_type=jnp.float32) + b2_ref[...]
    y = h + t
    m = jnp.mean(y, axis=-1, keepdims=True)
    v = jnp.mean((y - m) * (y - m), axis=-1, keepdims=True)
    o_ref[...] = (y - m) * lax.rsqrt(v + 1e-5) * g_ref[...] + b_ref[...]


def _dense_layer(h, p0, p1, W1, b1, W2, b2, g, b):
    blk = 2000
    return pl.pallas_call(
        _dense_body,
        grid=(_N // blk,),
        in_specs=[
            pl.BlockSpec((blk, _H), lambda i: (i, 0)),
            pl.BlockSpec((blk, _H), lambda i: (i, 0)),
            pl.BlockSpec((blk, _H), lambda i: (i, 0)),
            pl.BlockSpec((_H, _H), lambda i: (0, 0)),
            pl.BlockSpec((1, _H), lambda i: (0, 0)),
            pl.BlockSpec((_H, _H), lambda i: (0, 0)),
            pl.BlockSpec((1, _H), lambda i: (0, 0)),
            pl.BlockSpec((1, _H), lambda i: (0, 0)),
            pl.BlockSpec((1, _H), lambda i: (0, 0)),
        ],
        out_specs=pl.BlockSpec((blk, _H), lambda i: (i, 0)),
        out_shape=jax.ShapeDtypeStruct((_N, _H), jnp.float32),
    )(h, p0, p1, W1, b1.reshape(1, _H), W2, b2.reshape(1, _H),
      g.reshape(1, _H), b.reshape(1, _H))


def kernel(z, x, edge_index, edge_attr, batch_vec, emb, feat_W, feat_b,
           ln0_g, ln0_b, edge_W, edge_b, conv_W1, conv_b1, conv_W2, conv_b2,
           ln_g, ln_b):
    h = _node_init(z, x, emb, feat_W, feat_b, ln0_g, ln0_b)
    e = _edge_init(edge_attr, edge_W, edge_b)
    src = edge_index[0]
    dst = edge_index[1]
    for i in range(4):
        parts = _sc_message(h, e, src, dst)
        h = _dense_layer(h, parts[:_N], parts[_N:], conv_W1[i], conv_b1[i],
                         conv_W2[i], conv_b2[i], ln_g[i], ln_b[i])
    return (h, batch_vec)
